# Initial kernel scaffold; baseline (speedup 1.0000x reference)
import functools

import jax
import jax.numpy as jnp
from jax import lax
from jax.experimental import pallas as pl
from jax.experimental.pallas import tpu as pltpu
from jax.experimental.pallas import tpu_sc as plsc

N = 10000
D = 256
F = 16
C = 7
E = 160000

NC = 2
NS = 16
NW = NC * NS
CHUNK = 128
CPS = 40
E_PAD = NW * CPS * CHUNK
N_PAD = 10240
RPS = N_PAD // NS


def _sc_mesh():
    return plsc.VectorSubcoreMesh(core_axis_name="c", subcore_axis_name="s")


def _sc_hist(dst2d, ones, zeros):
    @functools.partial(
        pl.kernel,
        out_type=jax.ShapeDtypeStruct((NC, N_PAD, F), jnp.float32),
        mesh=_sc_mesh(),
        scratch_types=[
            pltpu.VMEM((CPS, CHUNK), jnp.int32),
            pltpu.VMEM((CHUNK, F), jnp.float32),
            pltpu.VMEM_SHARED((N_PAD, F), jnp.float32),
        ],
    )
    def k(dst_hbm, ones_hbm, z_hbm, out_hbm, didx, obuf, acc):
        cid = lax.axis_index("c")
        sid = lax.axis_index("s")
        wid = cid * NS + sid
        pltpu.sync_copy(z_hbm, acc.at[pl.ds(sid * RPS, RPS)])
        pltpu.sync_copy(ones_hbm, obuf)
        pltpu.sync_copy(dst_hbm.at[pl.ds(wid * CPS, CPS)], didx)
        plsc.subcore_barrier()

        @pl.loop(0, CPS)
        def _(j):
            pltpu.sync_copy(obuf, acc.at[didx.at[j]], add=True)

        plsc.subcore_barrier()
        pltpu.sync_copy(acc.at[pl.ds(sid * RPS, RPS)],
                        out_hbm.at[cid].at[pl.ds(sid * RPS, RPS)])

    return k(dst2d, ones, zeros)


def _sc_segsum(g, src2d, dst2d, zeros):
    @functools.partial(
        pl.kernel,
        out_type=jax.ShapeDtypeStruct((NC, N_PAD, F), jnp.float32),
        mesh=_sc_mesh(),
        scratch_types=[
            pltpu.VMEM((CPS, CHUNK), jnp.int32),
            pltpu.VMEM((CPS, CHUNK), jnp.int32),
            pltpu.VMEM((CHUNK, F), jnp.float32),
            pltpu.VMEM_SHARED((N_PAD, F), jnp.float32),
            pltpu.SemaphoreType.DMA,
        ],
    )
    def k(g_hbm, src_hbm, dst_hbm, z_hbm, out_hbm, sidx, didx, rows, acc, sem):
        cid = lax.axis_index("c")
        sid = lax.axis_index("s")
        wid = cid * NS + sid
        pltpu.sync_copy(z_hbm, acc.at[pl.ds(sid * RPS, RPS)])
        pltpu.sync_copy(src_hbm.at[pl.ds(wid * CPS, CPS)], sidx)
        pltpu.sync_copy(dst_hbm.at[pl.ds(wid * CPS, CPS)], didx)
        plsc.subcore_barrier()

        @pl.loop(0, CPS)
        def _(j):
            pltpu.async_copy(g_hbm.at[sidx.at[j]], rows, sem).wait()
            pltpu.sync_copy(rows, acc.at[didx.at[j]], add=True)

        plsc.subcore_barrier()
        pltpu.sync_copy(acc.at[pl.ds(sid * RPS, RPS)],
                        out_hbm.at[cid].at[pl.ds(sid * RPS, RPS)])

    return k(g, src2d, dst2d, zeros)


def _tc_mm1(x, W1):
    def body(x_ref, w_ref, o_ref):
        o_ref[...] = jnp.dot(x_ref[...], w_ref[...],
                             preferred_element_type=jnp.float32,
                             precision=lax.Precision.HIGHEST)

    return pl.pallas_call(
        body,
        out_shape=jax.ShapeDtypeStruct((N, F), jnp.float32),
        grid=(5,),
        in_specs=[pl.BlockSpec((N // 5, D), lambda i: (i, 0)),
                  pl.BlockSpec((D, F), lambda i: (0, 0))],
        out_specs=pl.BlockSpec((N // 5, F), lambda i: (i, 0)),
    )(x, W1)


def _tc_pre(counts, h1):
    def body(c_ref, h_ref, dis_ref, g_ref):
        deg = c_ref[0] + c_ref[1] + 1.0
        dis = 1.0 / jnp.sqrt(deg)
        dis_ref[...] = dis
        g_ref[...] = h_ref[...] * dis

    return pl.pallas_call(
        body,
        out_shape=(jax.ShapeDtypeStruct((N, F), jnp.float32),
                   jax.ShapeDtypeStruct((N, F), jnp.float32)),
    )(counts, h1)


def _tc_mid(s1, g1, dis, b1, W2p):
    def body(s_ref, g_ref, d_ref, b_ref, w_ref, h1_ref, g2_ref):
        pre = d_ref[...] * (s_ref[0] + s_ref[1] + g_ref[...]) + b_ref[...]
        H1 = jnp.maximum(pre, 0.0)
        h1_ref[...] = H1
        h2 = jnp.dot(H1, w_ref[...], preferred_element_type=jnp.float32,
                     precision=lax.Precision.HIGHEST)
        g2_ref[...] = h2 * d_ref[...]

    return pl.pallas_call(
        body,
        out_shape=(jax.ShapeDtypeStruct((N, F), jnp.float32),
                   jax.ShapeDtypeStruct((N, F), jnp.float32)),
    )(s1, g1, dis, b1, W2p)


def _tc_post(s2, g2, dis, b2p):
    def body(s_ref, g_ref, d_ref, b_ref, h2_ref, lp_ref):
        t = d_ref[...] * (s_ref[0] + s_ref[1] + g_ref[...]) + b_ref[...]
        h2_ref[...] = t
        r = jnp.maximum(t, 0.0)
        col = lax.broadcasted_iota(jnp.int32, (N, F), 1)
        rm = jnp.where(col < C, r, -jnp.inf)
        m = jnp.max(rm, axis=1, keepdims=True)
        lse = m + jnp.log(jnp.sum(jnp.exp(rm - m), axis=1, keepdims=True))
        lp_ref[...] = r - lse

    return pl.pallas_call(
        body,
        out_shape=(jax.ShapeDtypeStruct((N, F), jnp.float32),
                   jax.ShapeDtypeStruct((N, F), jnp.float32)),
    )(s2, g2, dis, b2p)


def kernel(x, edge_index, W1, b1, W2, b2):
    ei = edge_index.astype(jnp.int32)
    pad = E_PAD - E
    src2d = jnp.concatenate([ei[0], jnp.zeros((pad,), jnp.int32)]
                            ).reshape(E_PAD // CHUNK, CHUNK)
    dst2d = jnp.concatenate([ei[1], jnp.full((pad,), N, jnp.int32)]
                            ).reshape(E_PAD // CHUNK, CHUNK)
    zeros = jnp.zeros((RPS, F), jnp.float32)
    ones = jnp.ones((CHUNK, F), jnp.float32)
    W2p = jnp.pad(W2, ((0, 0), (0, F - C)))
    b1r = b1.reshape(1, F)
    b2p = jnp.pad(b2, (0, F - C)).reshape(1, F)

    counts = _sc_hist(dst2d, ones, zeros)
    h1 = _tc_mm1(x, W1)
    dis, g1 = _tc_pre(counts[:, :N, :], h1)
    s1 = _sc_segsum(g1, src2d, dst2d, zeros)
    H1, g2 = _tc_mid(s1[:, :N, :], g1, dis, b1r, W2p)
    s2 = _sc_segsum(g2, src2d, dst2d, zeros)
    H2p, lp = _tc_post(s2[:, :N, :], g2, dis, b2p)
    return (lp[:, :C], x, H1, H2p[:, :C])


# speedup vs baseline: 18.0166x; 18.0166x over previous
import functools

import jax
import jax.numpy as jnp
from jax import lax
from jax.experimental import pallas as pl
from jax.experimental.pallas import tpu as pltpu
from jax.experimental.pallas import tpu_sc as plsc

N = 10000
D = 256
F = 16
C = 7
E = 160000

NC = 2
NS = 16
NW = NC * NS
CHUNK = 128
CPS = 40
E_PAD = NW * CPS * CHUNK
N_PAD = 10240
RPS = N_PAD // NS


def _sc_mesh():
    return plsc.VectorSubcoreMesh(core_axis_name="c", subcore_axis_name="s")


_SC_PARAMS = pltpu.CompilerParams(use_tc_tiling_on_sc=False)


def _sc_hist(dst2d, ones, zeros):
    @functools.partial(
        pl.kernel,
        out_type=jax.ShapeDtypeStruct((NC, N_PAD, F), jnp.float32),
        mesh=_sc_mesh(),
        scratch_types=[
            pltpu.VMEM((CPS, CHUNK), jnp.int32),
            pltpu.VMEM((CHUNK, F), jnp.float32),
            pltpu.VMEM_SHARED((N_PAD, F), jnp.float32),
        ],
        compiler_params=_SC_PARAMS,
    )
    def k(dst_hbm, ones_hbm, z_hbm, out_hbm, didx, obuf, acc):
        cid = lax.axis_index("c")
        sid = lax.axis_index("s")
        wid = cid * NS + sid
        pltpu.sync_copy(z_hbm, acc.at[pl.ds(sid * RPS, RPS)])
        pltpu.sync_copy(ones_hbm, obuf)
        pltpu.sync_copy(dst_hbm.at[pl.ds(wid * CPS, CPS)], didx)
        plsc.subcore_barrier()

        @pl.loop(0, CPS)
        def _(j):
            pltpu.sync_copy(obuf, acc.at[didx.at[j]], add=True)

        plsc.subcore_barrier()
        pltpu.sync_copy(acc.at[pl.ds(sid * RPS, RPS)],
                        out_hbm.at[cid].at[pl.ds(sid * RPS, RPS)])

    return k(dst2d, ones, zeros)


def _sc_segsum(g, src2d, dst2d, zeros):
    @functools.partial(
        pl.kernel,
        out_type=jax.ShapeDtypeStruct((NC, N_PAD, F), jnp.float32),
        mesh=_sc_mesh(),
        scratch_types=[
            pltpu.VMEM((CPS, CHUNK), jnp.int32),
            pltpu.VMEM((CPS, CHUNK), jnp.int32),
            pltpu.VMEM((CHUNK, F), jnp.float32),
            pltpu.VMEM_SHARED((N_PAD, F), jnp.float32),
            pltpu.SemaphoreType.DMA,
        ],
        compiler_params=_SC_PARAMS,
    )
    def k(g_hbm, src_hbm, dst_hbm, z_hbm, out_hbm, sidx, didx, rows, acc, sem):
        cid = lax.axis_index("c")
        sid = lax.axis_index("s")
        wid = cid * NS + sid
        pltpu.sync_copy(z_hbm, acc.at[pl.ds(sid * RPS, RPS)])
        pltpu.sync_copy(src_hbm.at[pl.ds(wid * CPS, CPS)], sidx)
        pltpu.sync_copy(dst_hbm.at[pl.ds(wid * CPS, CPS)], didx)
        plsc.subcore_barrier()

        @pl.loop(0, CPS)
        def _(j):
            pltpu.async_copy(g_hbm.at[sidx.at[j]], rows, sem).wait()
            pltpu.sync_copy(rows, acc.at[didx.at[j]], add=True)

        plsc.subcore_barrier()
        pltpu.sync_copy(acc.at[pl.ds(sid * RPS, RPS)],
                        out_hbm.at[cid].at[pl.ds(sid * RPS, RPS)])

    return k(g, src2d, dst2d, zeros)


def _tc_mm1(x, W1):
    def body(x_ref, w_ref, o_ref):
        o_ref[...] = jnp.dot(x_ref[...], w_ref[...],
                             preferred_element_type=jnp.float32,
                             precision=lax.Precision.HIGHEST)

    return pl.pallas_call(
        body,
        out_shape=jax.ShapeDtypeStruct((N, F), jnp.float32),
        grid=(5,),
        in_specs=[pl.BlockSpec((N // 5, D), lambda i: (i, 0)),
                  pl.BlockSpec((D, F), lambda i: (0, 0))],
        out_specs=pl.BlockSpec((N // 5, F), lambda i: (i, 0)),
    )(x, W1)


def _tc_pre(counts, h1):
    def body(c_ref, h_ref, dis_ref, g_ref):
        deg = c_ref[0] + c_ref[1] + 1.0
        dis = 1.0 / jnp.sqrt(deg)
        dis_ref[...] = dis
        g_ref[...] = h_ref[...] * dis

    return pl.pallas_call(
        body,
        out_shape=(jax.ShapeDtypeStruct((N, F), jnp.float32),
                   jax.ShapeDtypeStruct((N, F), jnp.float32)),
    )(counts, h1)


def _tc_mid(s1, g1, dis, b1, W2p):
    def body(s_ref, g_ref, d_ref, b_ref, w_ref, h1_ref, g2_ref):
        pre = d_ref[...] * (s_ref[0] + s_ref[1] + g_ref[...]) + b_ref[...]
        H1 = jnp.maximum(pre, 0.0)
        h1_ref[...] = H1
        h2 = jnp.dot(H1, w_ref[...], preferred_element_type=jnp.float32,
                     precision=lax.Precision.HIGHEST)
        g2_ref[...] = h2 * d_ref[...]

    return pl.pallas_call(
        body,
        out_shape=(jax.ShapeDtypeStruct((N, F), jnp.float32),
                   jax.ShapeDtypeStruct((N, F), jnp.float32)),
    )(s1, g1, dis, b1, W2p)


def _tc_post(s2, g2, dis, b2p):
    def body(s_ref, g_ref, d_ref, b_ref, h2_ref, lp_ref):
        t = d_ref[...] * (s_ref[0] + s_ref[1] + g_ref[...]) + b_ref[...]
        h2_ref[...] = t
        r = jnp.maximum(t, 0.0)
        col = lax.broadcasted_iota(jnp.int32, (N, F), 1)
        rm = jnp.where(col < C, r, -jnp.inf)
        m = jnp.max(rm, axis=1, keepdims=True)
        lse = m + jnp.log(jnp.sum(jnp.exp(rm - m), axis=1, keepdims=True))
        lp_ref[...] = r - lse

    return pl.pallas_call(
        body,
        out_shape=(jax.ShapeDtypeStruct((N, F), jnp.float32),
                   jax.ShapeDtypeStruct((N, F), jnp.float32)),
    )(s2, g2, dis, b2p)


def kernel(x, edge_index, W1, b1, W2, b2):
    ei = edge_index.astype(jnp.int32)
    pad = E_PAD - E
    src2d = jnp.concatenate([ei[0], jnp.zeros((pad,), jnp.int32)]
                            ).reshape(E_PAD // CHUNK, CHUNK)
    dst2d = jnp.concatenate([ei[1], jnp.full((pad,), N, jnp.int32)]
                            ).reshape(E_PAD // CHUNK, CHUNK)
    zeros = jnp.zeros((RPS, F), jnp.float32)
    ones = jnp.ones((CHUNK, F), jnp.float32)
    W2p = jnp.pad(W2, ((0, 0), (0, F - C)))
    b1r = b1.reshape(1, F)
    b2p = jnp.pad(b2, (0, F - C)).reshape(1, F)

    counts = _sc_hist(dst2d, ones, zeros)
    h1 = _tc_mm1(x, W1)
    dis, g1 = _tc_pre(counts[:, :N, :], h1)
    s1 = _sc_segsum(g1, src2d, dst2d, zeros)
    H1, g2 = _tc_mid(s1[:, :N, :], g1, dis, b1r, W2p)
    s2 = _sc_segsum(g2, src2d, dst2d, zeros)
    H2p, lp = _tc_post(s2[:, :N, :], g2, dis, b2p)
    return (lp[:, :C], x, H1, H2p[:, :C])


# speedup vs baseline: 20.1103x; 1.1162x over previous
import functools

import jax
import jax.numpy as jnp
from jax import lax
from jax.experimental import pallas as pl
from jax.experimental.pallas import tpu as pltpu
from jax.experimental.pallas import tpu_sc as plsc

N = 10000
D = 256
F = 16
C = 7
E = 160000

NC = 2
NS = 16
NW = NC * NS
CHUNK = 128
CPS = 40
E_PAD = NW * CPS * CHUNK
N_PAD = 10240
RPS = N_PAD // NS
HALF = CPS // 2


def _sc_mesh():
    return plsc.VectorSubcoreMesh(core_axis_name="c", subcore_axis_name="s")


_SC_PARAMS = pltpu.CompilerParams(use_tc_tiling_on_sc=False)


def _sc_hist(dst2d, ones, zeros):
    @functools.partial(
        pl.kernel,
        out_type=jax.ShapeDtypeStruct((NC, N_PAD, F), jnp.float32),
        mesh=_sc_mesh(),
        scratch_types=[
            pltpu.VMEM((CPS, CHUNK), jnp.int32),
            pltpu.VMEM((CHUNK, F), jnp.float32),
            pltpu.VMEM_SHARED((N_PAD, F), jnp.float32),
        ],
        compiler_params=_SC_PARAMS,
    )
    def k(dst_hbm, ones_hbm, z_hbm, out_hbm, didx, obuf, acc):
        cid = lax.axis_index("c")
        sid = lax.axis_index("s")
        wid = cid * NS + sid
        pltpu.sync_copy(z_hbm, acc.at[pl.ds(sid * RPS, RPS)])
        pltpu.sync_copy(ones_hbm, obuf)
        pltpu.sync_copy(dst_hbm.at[pl.ds(wid * CPS, CPS)], didx)
        plsc.subcore_barrier()

        @pl.loop(0, CPS)
        def _(j):
            pltpu.sync_copy(obuf, acc.at[didx.at[j]], add=True)

        plsc.subcore_barrier()
        pltpu.sync_copy(acc.at[pl.ds(sid * RPS, RPS)],
                        out_hbm.at[cid].at[pl.ds(sid * RPS, RPS)])

    return k(dst2d, ones, zeros)


def _sc_segsum(g, src2d, dst2d, zeros):
    @functools.partial(
        pl.kernel,
        out_type=jax.ShapeDtypeStruct((NC, N_PAD, F), jnp.float32),
        mesh=_sc_mesh(),
        scratch_types=[
            pltpu.VMEM((CPS, CHUNK), jnp.int32),
            pltpu.VMEM((CPS, CHUNK), jnp.int32),
            pltpu.VMEM((CHUNK, F), jnp.float32),
            pltpu.VMEM((CHUNK, F), jnp.float32),
            pltpu.VMEM_SHARED((N_PAD, F), jnp.float32),
            pltpu.SemaphoreType.DMA,
            pltpu.SemaphoreType.DMA,
        ],
        compiler_params=_SC_PARAMS,
    )
    def k(g_hbm, src_hbm, dst_hbm, z_hbm, out_hbm, sidx, didx, rows_a, rows_b,
          acc, sem_a, sem_b):
        cid = lax.axis_index("c")
        sid = lax.axis_index("s")
        wid = cid * NS + sid
        pltpu.sync_copy(z_hbm, acc.at[pl.ds(sid * RPS, RPS)])
        pltpu.sync_copy(src_hbm.at[pl.ds(wid * CPS, CPS)], sidx)
        pltpu.sync_copy(dst_hbm.at[pl.ds(wid * CPS, CPS)], didx)
        plsc.subcore_barrier()
        pltpu.async_copy(g_hbm.at[sidx.at[0]], rows_a, sem_a)

        @pl.loop(0, CPS, step=2)
        def _(j):
            pltpu.async_copy(g_hbm.at[sidx.at[j + 1]], rows_b, sem_b)
            pltpu.make_async_copy(g_hbm.at[sidx.at[j]], rows_a, sem_a).wait()
            pltpu.sync_copy(rows_a, acc.at[didx.at[j]], add=True)

            @pl.when(j + 2 < CPS)
            def _():
                pltpu.async_copy(g_hbm.at[sidx.at[j + 2]], rows_a, sem_a)

            pltpu.make_async_copy(g_hbm.at[sidx.at[j + 1]], rows_b, sem_b).wait()
            pltpu.sync_copy(rows_b, acc.at[didx.at[j + 1]], add=True)

        plsc.subcore_barrier()
        pltpu.sync_copy(acc.at[pl.ds(sid * RPS, RPS)],
                        out_hbm.at[cid].at[pl.ds(sid * RPS, RPS)])

    return k(g, src2d, dst2d, zeros)


def _tc_l1(counts, x, W1):
    def body(c_ref, x_ref, w_ref, dis_ref, g_ref):
        deg = c_ref[0] + c_ref[1] + 1.0
        dis = 1.0 / jnp.sqrt(deg)
        dis_ref[...] = dis
        h1 = jnp.dot(x_ref[...], w_ref[...],
                     preferred_element_type=jnp.float32,
                     precision=lax.Precision.HIGHEST)
        g_ref[...] = h1 * dis

    B = N // 5
    return pl.pallas_call(
        body,
        out_shape=(jax.ShapeDtypeStruct((N, F), jnp.float32),
                   jax.ShapeDtypeStruct((N, F), jnp.float32)),
        grid=(5,),
        in_specs=[pl.BlockSpec((2, B, F), lambda i: (0, i, 0)),
                  pl.BlockSpec((B, D), lambda i: (i, 0)),
                  pl.BlockSpec((D, F), lambda i: (0, 0))],
        out_specs=(pl.BlockSpec((B, F), lambda i: (i, 0)),
                   pl.BlockSpec((B, F), lambda i: (i, 0))),
    )(counts, x, W1)


def _tc_mid(s1, g1, dis, b1, W2p):
    def body(s_ref, g_ref, d_ref, b_ref, w_ref, h1_ref, g2_ref):
        pre = d_ref[...] * (s_ref[0] + s_ref[1] + g_ref[...]) + b_ref[...]
        H1 = jnp.maximum(pre, 0.0)
        h1_ref[...] = H1
        h2 = jnp.dot(H1, w_ref[...], preferred_element_type=jnp.float32,
                     precision=lax.Precision.HIGHEST)
        g2_ref[...] = h2 * d_ref[...]

    return pl.pallas_call(
        body,
        out_shape=(jax.ShapeDtypeStruct((N, F), jnp.float32),
                   jax.ShapeDtypeStruct((N, F), jnp.float32)),
    )(s1, g1, dis, b1, W2p)


def _tc_post(s2, g2, dis, b2p):
    def body(s_ref, g_ref, d_ref, b_ref, h2_ref, lp_ref):
        t = d_ref[...] * (s_ref[0] + s_ref[1] + g_ref[...]) + b_ref[...]
        h2_ref[...] = t
        r = jnp.maximum(t, 0.0)
        col = lax.broadcasted_iota(jnp.int32, (N, F), 1)
        rm = jnp.where(col < C, r, -jnp.inf)
        m = jnp.max(rm, axis=1, keepdims=True)
        lse = m + jnp.log(jnp.sum(jnp.exp(rm - m), axis=1, keepdims=True))
        lp_ref[...] = r - lse

    return pl.pallas_call(
        body,
        out_shape=(jax.ShapeDtypeStruct((N, F), jnp.float32),
                   jax.ShapeDtypeStruct((N, F), jnp.float32)),
    )(s2, g2, dis, b2p)


def kernel(x, edge_index, W1, b1, W2, b2):
    ei = edge_index.astype(jnp.int32)
    pad = E_PAD - E
    src2d = jnp.concatenate([ei[0], jnp.zeros((pad,), jnp.int32)]
                            ).reshape(E_PAD // CHUNK, CHUNK)
    dst2d = jnp.concatenate([ei[1], jnp.full((pad,), N, jnp.int32)]
                            ).reshape(E_PAD // CHUNK, CHUNK)
    zeros = jnp.zeros((RPS, F), jnp.float32)
    ones = jnp.ones((CHUNK, F), jnp.float32)
    W2p = jnp.pad(W2, ((0, 0), (0, F - C)))
    b1r = b1.reshape(1, F)
    b2p = jnp.pad(b2, (0, F - C)).reshape(1, F)

    counts = _sc_hist(dst2d, ones, zeros)
    dis, g1 = _tc_l1(counts[:, :N, :], x, W1)

    s1 = _sc_segsum(g1, src2d, dst2d, zeros)
    H1, g2 = _tc_mid(s1[:, :N, :], g1, dis, b1r, W2p)
    s2 = _sc_segsum(g2, src2d, dst2d, zeros)
    H2p, lp = _tc_post(s2[:, :N, :], g2, dis, b2p)
    return (lp[:, :C], x, H1, H2p[:, :C])


# speedup vs baseline: 23.1000x; 1.1487x over previous
import functools

import jax
import jax.numpy as jnp
from jax import lax
from jax.experimental import pallas as pl
from jax.experimental.pallas import tpu as pltpu
from jax.experimental.pallas import tpu_sc as plsc

N = 10000
D = 256
F = 16
C = 7
E = 160000

NC = 2
NS = 16
NW = NC * NS
CHUNK = 128
CPS = 40
E_ROWS = E // CHUNK
REAL_ROWS_LAST = E_ROWS - (NW - 1) * CPS
PAD_ROWS = CPS - REAL_ROWS_LAST
N_PAD = 10240
RPS = N_PAD // NS
HALF = CPS // 2
GB = 5


def _sc_mesh():
    return plsc.VectorSubcoreMesh(core_axis_name="c", subcore_axis_name="s")


_SC_PARAMS = pltpu.CompilerParams(use_tc_tiling_on_sc=False)


def _load_idx(idx_hbm, pad_hbm, idx_vmem, wid):
    @pl.when(wid < NW - 1)
    def _():
        pltpu.sync_copy(idx_hbm.at[pl.ds(wid * CPS, CPS)], idx_vmem)

    @pl.when(wid == NW - 1)
    def _():
        pltpu.sync_copy(idx_hbm.at[pl.ds((NW - 1) * CPS, REAL_ROWS_LAST)],
                        idx_vmem.at[pl.ds(0, REAL_ROWS_LAST)])
        pltpu.sync_copy(pad_hbm, idx_vmem.at[pl.ds(REAL_ROWS_LAST, PAD_ROWS)])


def _sc_hist(dst2d, dpad, ones, zeros):
    @functools.partial(
        pl.kernel,
        out_type=jax.ShapeDtypeStruct((NC, N_PAD, F), jnp.float32),
        mesh=_sc_mesh(),
        scratch_types=[
            pltpu.VMEM((CPS, CHUNK), jnp.int32),
            pltpu.VMEM((CHUNK, F), jnp.float32),
            pltpu.VMEM_SHARED((N_PAD, F), jnp.float32),
        ],
        compiler_params=_SC_PARAMS,
    )
    def k(dst_hbm, dpad_hbm, ones_hbm, z_hbm, out_hbm, didx, obuf, acc):
        cid = lax.axis_index("c")
        sid = lax.axis_index("s")
        wid = cid * NS + sid
        pltpu.sync_copy(z_hbm, acc.at[pl.ds(sid * RPS, RPS)])
        pltpu.sync_copy(ones_hbm, obuf)
        _load_idx(dst_hbm, dpad_hbm, didx, wid)
        plsc.subcore_barrier()

        @pl.loop(0, CPS)
        def _(j):
            pltpu.sync_copy(obuf, acc.at[didx.at[j]], add=True)

        plsc.subcore_barrier()
        pltpu.sync_copy(acc.at[pl.ds(sid * RPS, RPS)],
                        out_hbm.at[cid].at[pl.ds(sid * RPS, RPS)])

    return k(dst2d, dpad, ones, zeros)


def _sc_segsum(g, src2d, dst2d, spad, dpad, zeros):
    @functools.partial(
        pl.kernel,
        out_type=jax.ShapeDtypeStruct((NC, N_PAD, F), jnp.float32),
        mesh=_sc_mesh(),
        scratch_types=[
            pltpu.VMEM((CPS, CHUNK), jnp.int32),
            pltpu.VMEM((CPS, CHUNK), jnp.int32),
            pltpu.VMEM((CHUNK, F), jnp.float32),
            pltpu.VMEM((CHUNK, F), jnp.float32),
            pltpu.VMEM_SHARED((N_PAD, F), jnp.float32),
            pltpu.SemaphoreType.DMA,
            pltpu.SemaphoreType.DMA,
        ],
        compiler_params=_SC_PARAMS,
    )
    def k(g_hbm, src_hbm, dst_hbm, spad_hbm, dpad_hbm, z_hbm, out_hbm,
          sidx, didx, rows_a, rows_b, acc, sem_a, sem_b):
        cid = lax.axis_index("c")
        sid = lax.axis_index("s")
        wid = cid * NS + sid
        pltpu.sync_copy(z_hbm, acc.at[pl.ds(sid * RPS, RPS)])
        _load_idx(src_hbm, spad_hbm, sidx, wid)
        _load_idx(dst_hbm, dpad_hbm, didx, wid)
        plsc.subcore_barrier()
        pltpu.async_copy(g_hbm.at[sidx.at[0]], rows_a, sem_a)

        @pl.loop(0, CPS, step=2)
        def _(j):
            pltpu.async_copy(g_hbm.at[sidx.at[j + 1]], rows_b, sem_b)
            pltpu.make_async_copy(g_hbm.at[sidx.at[j]], rows_a, sem_a).wait()
            pltpu.sync_copy(rows_a, acc.at[didx.at[j]], add=True)

            @pl.when(j + 2 < CPS)
            def _():
                pltpu.async_copy(g_hbm.at[sidx.at[j + 2]], rows_a, sem_a)

            pltpu.make_async_copy(g_hbm.at[sidx.at[j + 1]], rows_b, sem_b).wait()
            pltpu.sync_copy(rows_b, acc.at[didx.at[j + 1]], add=True)

        plsc.subcore_barrier()
        pltpu.sync_copy(acc.at[pl.ds(sid * RPS, RPS)],
                        out_hbm.at[cid].at[pl.ds(sid * RPS, RPS)])

    return k(g, src2d, dst2d, spad, dpad, zeros)


def _tc_l1(counts, x, W1):
    def body(c_ref, x_ref, w_ref, dis_ref, g_ref):
        deg = c_ref[0] + c_ref[1] + 1.0
        dis = 1.0 / jnp.sqrt(deg)
        dis_ref[...] = dis
        h1 = jnp.dot(x_ref[...], w_ref[...],
                     preferred_element_type=jnp.float32,
                     precision=lax.Precision.HIGHEST)
        g_ref[...] = h1 * dis

    B = N // GB
    return pl.pallas_call(
        body,
        out_shape=(jax.ShapeDtypeStruct((N, F), jnp.float32),
                   jax.ShapeDtypeStruct((N, F), jnp.float32)),
        grid=(GB,),
        in_specs=[pl.BlockSpec((2, B, F), lambda i: (0, i, 0)),
                  pl.BlockSpec((B, D), lambda i: (i, 0)),
                  pl.BlockSpec((D, F), lambda i: (0, 0))],
        out_specs=(pl.BlockSpec((B, F), lambda i: (i, 0)),
                   pl.BlockSpec((B, F), lambda i: (i, 0))),
    )(counts, x, W1)


def _tc_mid(s1, g1, dis, b1, W2p):
    def body(s_ref, g_ref, d_ref, b_ref, w_ref, h1_ref, g2_ref):
        pre = d_ref[...] * (s_ref[0] + s_ref[1] + g_ref[...]) + b_ref[...]
        H1 = jnp.maximum(pre, 0.0)
        h1_ref[...] = H1
        h2 = jnp.dot(H1, w_ref[...], preferred_element_type=jnp.float32,
                     precision=lax.Precision.HIGHEST)
        g2_ref[...] = h2 * d_ref[...]

    B = N // GB
    return pl.pallas_call(
        body,
        out_shape=(jax.ShapeDtypeStruct((N, F), jnp.float32),
                   jax.ShapeDtypeStruct((N, F), jnp.float32)),
        grid=(GB,),
        in_specs=[pl.BlockSpec((2, B, F), lambda i: (0, i, 0)),
                  pl.BlockSpec((B, F), lambda i: (i, 0)),
                  pl.BlockSpec((B, F), lambda i: (i, 0)),
                  pl.BlockSpec((1, F), lambda i: (0, 0)),
                  pl.BlockSpec((F, F), lambda i: (0, 0))],
        out_specs=(pl.BlockSpec((B, F), lambda i: (i, 0)),
                   pl.BlockSpec((B, F), lambda i: (i, 0))),
    )(s1, g1, dis, b1, W2p)


def _tc_post(s2, g2, dis, b2p):
    def body(s_ref, g_ref, d_ref, b_ref, h2_ref, lp_ref):
        t = d_ref[...] * (s_ref[0] + s_ref[1] + g_ref[...]) + b_ref[...]
        h2_ref[...] = t
        r = jnp.maximum(t, 0.0)
        col = lax.broadcasted_iota(jnp.int32, r.shape, 1)
        rm = jnp.where(col < C, r, -jnp.inf)
        m = jnp.max(rm, axis=1, keepdims=True)
        lse = m + jnp.log(jnp.sum(jnp.exp(rm - m), axis=1, keepdims=True))
        lp_ref[...] = r - lse

    B = N // GB
    return pl.pallas_call(
        body,
        out_shape=(jax.ShapeDtypeStruct((N, F), jnp.float32),
                   jax.ShapeDtypeStruct((N, F), jnp.float32)),
        grid=(GB,),
        in_specs=[pl.BlockSpec((2, B, F), lambda i: (0, i, 0)),
                  pl.BlockSpec((B, F), lambda i: (i, 0)),
                  pl.BlockSpec((B, F), lambda i: (i, 0)),
                  pl.BlockSpec((1, F), lambda i: (0, 0))],
        out_specs=(pl.BlockSpec((B, F), lambda i: (i, 0)),
                   pl.BlockSpec((B, F), lambda i: (i, 0))),
    )(s2, g2, dis, b2p)


def kernel(x, edge_index, W1, b1, W2, b2):
    ei = edge_index.astype(jnp.int32)
    src2d = ei[0].reshape(E_ROWS, CHUNK)
    dst2d = ei[1].reshape(E_ROWS, CHUNK)
    spad = jnp.zeros((PAD_ROWS, CHUNK), jnp.int32)
    dpad = jnp.full((PAD_ROWS, CHUNK), N, jnp.int32)
    zeros = jnp.zeros((RPS, F), jnp.float32)
    ones = jnp.ones((CHUNK, F), jnp.float32)
    W2p = jnp.pad(W2, ((0, 0), (0, F - C)))
    b1r = b1.reshape(1, F)
    b2p = jnp.pad(b2, (0, F - C)).reshape(1, F)

    counts = _sc_hist(dst2d, dpad, ones, zeros)
    dis, g1 = _tc_l1(counts, x, W1)
    s1 = _sc_segsum(g1, src2d, dst2d, spad, dpad, zeros)
    H1, g2 = _tc_mid(s1, g1, dis, b1r, W2p)
    s2 = _sc_segsum(g2, src2d, dst2d, spad, dpad, zeros)
    H2p, lp = _tc_post(s2, g2, dis, b2p)
    return (lp[:, :C], x, H1, H2p[:, :C])
